# baseline (speedup 1.0000x reference)
import functools

import jax
import jax.numpy as jnp
from jax.experimental import pallas as pl
from jax.experimental.pallas import tpu as pltpu

_QT = 256
_BK = 2048
_TOPK = 5
_NEG = -3.0e38


def _topk_kernel(qn_ref, bn_ref, vals_ref, idx_ref, rv_ref, rn_ref, *, nk, K):
    k = pl.program_id(1)

    @pl.when(k == 0)
    def _init():
        rv_ref[...] = jnp.full(rv_ref.shape, _NEG, jnp.float32)
        rn_ref[...] = jnp.full(rn_ref.shape, _NEG, jnp.float32)

    s = jax.lax.dot_general(qn_ref[...], bn_ref[...], (((1,), (1,)), ((), ())),
                            preferred_element_type=jnp.float32)
    col = jax.lax.broadcasted_iota(jnp.int32, s.shape, 1) + k * _BK
    negcol = -col.astype(jnp.float32)
    s = jnp.where(negcol > float(-K), s, _NEG)

    vcols, ncols = [], []
    for j in range(_TOPK):
        m = jnp.max(s, axis=1, keepdims=True)
        z = jnp.where(s == m, negcol, _NEG)
        ci = jnp.max(z, axis=1, keepdims=True)
        vcols.append(m)
        ncols.append(ci)
        if j < _TOPK - 1:
            s = jnp.where(z == ci, _NEG, s)

    cv = jnp.concatenate(vcols + [rv_ref[:, :_TOPK]], axis=1)
    cn = jnp.concatenate(ncols + [rn_ref[:, :_TOPK]], axis=1)
    nv, nn = [], []
    for j in range(_TOPK):
        m = jnp.max(cv, axis=1, keepdims=True)
        z = jnp.where(cv == m, cn, _NEG)
        ci = jnp.max(z, axis=1, keepdims=True)
        nv.append(m)
        nn.append(ci)
        cv = jnp.where(z == ci, _NEG, cv)
    pad = rv_ref.shape[1] - _TOPK
    fill = jnp.full((cv.shape[0], pad), _NEG, jnp.float32)
    rv_ref[...] = jnp.concatenate(nv + [fill], axis=1)
    rn_ref[...] = jnp.concatenate(nn + [fill], axis=1)

    @pl.when(k == nk - 1)
    def _emit():
        vals_ref[...] = rv_ref[:, :_TOPK]
        idx_ref[...] = (-rn_ref[:, :_TOPK]).astype(jnp.int32)


def _layer_norm(x, g, b, eps=1e-5):
    m = jnp.mean(x, axis=-1, keepdims=True)
    v = jnp.var(x, axis=-1, keepdims=True)
    return (x - m) / jnp.sqrt(v + eps) * g + b


def kernel(query, bank, W1, b1, g1, beta1, W2, b2, g2, beta2, top_k):
    Q, D = query.shape
    K = bank.shape[0]
    nq = Q // _QT
    nk = pl.cdiv(K, _BK)

    h = query @ W1 + b1
    h = _layer_norm(h, g1, beta1)
    h = jax.nn.relu(h)
    h = h @ W2 + b2
    q_emb = _layer_norm(h, g2, beta2)
    qn = q_emb / (jnp.linalg.norm(q_emb, axis=-1, keepdims=True) + 1e-8)
    bn = bank / (jnp.linalg.norm(bank, axis=-1, keepdims=True) + 1e-8)

    vals, idx = pl.pallas_call(
        functools.partial(_topk_kernel, nk=nk, K=K),
        grid=(nq, nk),
        in_specs=[
            pl.BlockSpec((_QT, D), lambda i, k: (i, 0)),
            pl.BlockSpec((_BK, D), lambda i, k: (k, 0)),
        ],
        out_specs=[
            pl.BlockSpec((_QT, _TOPK), lambda i, k: (i, 0)),
            pl.BlockSpec((_QT, _TOPK), lambda i, k: (i, 0)),
        ],
        out_shape=[
            jax.ShapeDtypeStruct((Q, _TOPK), jnp.float32),
            jax.ShapeDtypeStruct((Q, _TOPK), jnp.int32),
        ],
        scratch_shapes=[
            pltpu.VMEM((_QT, 128), jnp.float32),
            pltpu.VMEM((_QT, 128), jnp.float32),
        ],
        compiler_params=pltpu.CompilerParams(
            dimension_semantics=("parallel", "arbitrary"),
        ),
    )(qn, bn)
    return vals, idx


# speedup vs baseline: 1.0752x; 1.0752x over previous
import functools

import jax
import jax.numpy as jnp
from jax.experimental import pallas as pl
from jax.experimental.pallas import tpu as pltpu

_QT = 256
_BK = 2048
_TOPK = 5
_NEG = -3.0e38


def _topk_kernel(qn_ref, bank_ref, nrm_ref, vals_ref, idx_ref,
                 rv_ref, rn_ref, *, nk, K):
    k = pl.program_id(0)
    i = pl.program_id(1)
    rows = pl.ds(i * _QT, _QT)

    @pl.when(k == 0)
    def _init():
        rv_ref[rows, :] = jnp.full((_QT, rv_ref.shape[1]), _NEG, jnp.float32)
        rn_ref[rows, :] = jnp.full((_QT, rn_ref.shape[1]), _NEG, jnp.float32)

    bn = bank_ref[...] / nrm_ref[...]
    s = jax.lax.dot_general(qn_ref[...], bn, (((1,), (1,)), ((), ())),
                            preferred_element_type=jnp.float32)
    col = jax.lax.broadcasted_iota(jnp.int32, s.shape, 1) + k * _BK
    negcol = -col.astype(jnp.float32)
    s = jnp.where(col < K, s, _NEG)

    vcols, ncols = [], []
    for j in range(_TOPK):
        m = jnp.max(s, axis=1, keepdims=True)
        z = jnp.where(s == m, negcol, _NEG)
        ci = jnp.max(z, axis=1, keepdims=True)
        vcols.append(m)
        ncols.append(ci)
        if j < _TOPK - 1:
            s = jnp.where(z == ci, _NEG, s)

    cv = jnp.concatenate(vcols + [rv_ref[rows, :_TOPK]], axis=1)
    cn = jnp.concatenate(ncols + [rn_ref[rows, :_TOPK]], axis=1)
    nv, nn = [], []
    for j in range(_TOPK):
        m = jnp.max(cv, axis=1, keepdims=True)
        z = jnp.where(cv == m, cn, _NEG)
        ci = jnp.max(z, axis=1, keepdims=True)
        nv.append(m)
        nn.append(ci)
        cv = jnp.where(z == ci, _NEG, cv)
    pad = rv_ref.shape[1] - _TOPK
    fill = jnp.full((_QT, pad), _NEG, jnp.float32)
    rv_ref[rows, :] = jnp.concatenate(nv + [fill], axis=1)
    rn_ref[rows, :] = jnp.concatenate(nn + [fill], axis=1)

    @pl.when(k == nk - 1)
    def _emit():
        vals_ref[...] = rv_ref[rows, :_TOPK]
        idx_ref[...] = (-rn_ref[rows, :_TOPK]).astype(jnp.int32)


def _layer_norm(x, g, b, eps=1e-5):
    m = jnp.mean(x, axis=-1, keepdims=True)
    v = jnp.var(x, axis=-1, keepdims=True)
    return (x - m) / jnp.sqrt(v + eps) * g + b


def kernel(query, bank, W1, b1, g1, beta1, W2, b2, g2, beta2, top_k):
    Q, D = query.shape
    K = bank.shape[0]
    nq = Q // _QT
    nk = pl.cdiv(K, _BK)

    h = query @ W1 + b1
    h = _layer_norm(h, g1, beta1)
    h = jax.nn.relu(h)
    h = h @ W2 + b2
    q_emb = _layer_norm(h, g2, beta2)
    qn = q_emb / (jnp.linalg.norm(q_emb, axis=-1, keepdims=True) + 1e-8)
    nrm = jnp.linalg.norm(bank, axis=-1, keepdims=True) + 1e-8

    vals, idx = pl.pallas_call(
        functools.partial(_topk_kernel, nk=nk, K=K),
        grid=(nk, nq),
        in_specs=[
            pl.BlockSpec((_QT, D), lambda k, i: (i, 0)),
            pl.BlockSpec((_BK, D), lambda k, i: (k, 0)),
            pl.BlockSpec((_BK, 1), lambda k, i: (k, 0)),
        ],
        out_specs=[
            pl.BlockSpec((_QT, _TOPK), lambda k, i: (i, 0)),
            pl.BlockSpec((_QT, _TOPK), lambda k, i: (i, 0)),
        ],
        out_shape=[
            jax.ShapeDtypeStruct((Q, _TOPK), jnp.float32),
            jax.ShapeDtypeStruct((Q, _TOPK), jnp.int32),
        ],
        scratch_shapes=[
            pltpu.VMEM((Q, 128), jnp.float32),
            pltpu.VMEM((Q, 128), jnp.float32),
        ],
        compiler_params=pltpu.CompilerParams(
            dimension_semantics=("arbitrary", "arbitrary"),
        ),
    )(qn, bank, nrm)
    return vals, idx


# speedup vs baseline: 1.0904x; 1.0141x over previous
import functools

import jax
import jax.numpy as jnp
from jax.experimental import pallas as pl
from jax.experimental.pallas import tpu as pltpu

_QT = 256
_BK = 2048
_TOPK = 5
_NEG = -3.0e38


def _topk_kernel(qn_ref, bank_ref, nrm_ref, vals_ref, idx_ref,
                 rv_ref, rn_ref, *, nk, K):
    k = pl.program_id(0)
    i = pl.program_id(1)
    rows = pl.ds(i * _QT, _QT)

    @pl.when(k == 0)
    def _init():
        rv_ref[rows, :] = jnp.full((_QT, rv_ref.shape[1]), _NEG, jnp.float32)
        rn_ref[rows, :] = jnp.full((_QT, rn_ref.shape[1]), _NEG, jnp.float32)

    bn = bank_ref[...] / nrm_ref[...]
    s = jax.lax.dot_general(qn_ref[...], bn, (((1,), (1,)), ((), ())),
                            preferred_element_type=jnp.float32)
    col = jax.lax.broadcasted_iota(jnp.int32, s.shape, 1) + k * _BK
    negcol = -col.astype(jnp.float32)
    s = jnp.where(col < K, s, _NEG)

    nsl = s.shape[1] // 128
    c0 = negcol[:, 0:128]
    vcols, ncols = [], []
    for j in range(_TOPK):
        V = s[:, 0:128]
        C = c0
        for t in range(1, nsl):
            st = s[:, t * 128:(t + 1) * 128]
            ct = c0 - float(128 * t)
            g = st > V
            V = jnp.where(g, st, V)
            C = jnp.where(g, ct, C)
        m = jnp.max(V, axis=1, keepdims=True)
        z = jnp.where(V == m, C, _NEG)
        ci = jnp.max(z, axis=1, keepdims=True)
        vcols.append(m)
        ncols.append(ci)
        if j < _TOPK - 1:
            s = jnp.where(negcol == ci, _NEG, s)

    cv = jnp.concatenate(vcols + [rv_ref[rows, :_TOPK]], axis=1)
    cn = jnp.concatenate(ncols + [rn_ref[rows, :_TOPK]], axis=1)
    nv, nn = [], []
    for j in range(_TOPK):
        m = jnp.max(cv, axis=1, keepdims=True)
        z = jnp.where(cv == m, cn, _NEG)
        ci = jnp.max(z, axis=1, keepdims=True)
        nv.append(m)
        nn.append(ci)
        cv = jnp.where(z == ci, _NEG, cv)
    pad = rv_ref.shape[1] - _TOPK
    fill = jnp.full((_QT, pad), _NEG, jnp.float32)
    rv_ref[rows, :] = jnp.concatenate(nv + [fill], axis=1)
    rn_ref[rows, :] = jnp.concatenate(nn + [fill], axis=1)

    @pl.when(k == nk - 1)
    def _emit():
        vals_ref[...] = rv_ref[rows, :_TOPK]
        idx_ref[...] = (-rn_ref[rows, :_TOPK]).astype(jnp.int32)


def _layer_norm(x, g, b, eps=1e-5):
    m = jnp.mean(x, axis=-1, keepdims=True)
    v = jnp.var(x, axis=-1, keepdims=True)
    return (x - m) / jnp.sqrt(v + eps) * g + b


def kernel(query, bank, W1, b1, g1, beta1, W2, b2, g2, beta2, top_k):
    Q, D = query.shape
    K = bank.shape[0]
    nq = Q // _QT
    nk = pl.cdiv(K, _BK)

    h = query @ W1 + b1
    h = _layer_norm(h, g1, beta1)
    h = jax.nn.relu(h)
    h = h @ W2 + b2
    q_emb = _layer_norm(h, g2, beta2)
    qn = q_emb / (jnp.linalg.norm(q_emb, axis=-1, keepdims=True) + 1e-8)
    nrm = jnp.linalg.norm(bank, axis=-1, keepdims=True) + 1e-8

    vals, idx = pl.pallas_call(
        functools.partial(_topk_kernel, nk=nk, K=K),
        grid=(nk, nq),
        in_specs=[
            pl.BlockSpec((_QT, D), lambda k, i: (i, 0)),
            pl.BlockSpec((_BK, D), lambda k, i: (k, 0)),
            pl.BlockSpec((_BK, 1), lambda k, i: (k, 0)),
        ],
        out_specs=[
            pl.BlockSpec((_QT, _TOPK), lambda k, i: (i, 0)),
            pl.BlockSpec((_QT, _TOPK), lambda k, i: (i, 0)),
        ],
        out_shape=[
            jax.ShapeDtypeStruct((Q, _TOPK), jnp.float32),
            jax.ShapeDtypeStruct((Q, _TOPK), jnp.int32),
        ],
        scratch_shapes=[
            pltpu.VMEM((Q, 128), jnp.float32),
            pltpu.VMEM((Q, 128), jnp.float32),
        ],
        compiler_params=pltpu.CompilerParams(
            dimension_semantics=("arbitrary", "arbitrary"),
        ),
    )(qn, bank, nrm)
    return vals, idx


# speedup vs baseline: 1.1035x; 1.0120x over previous
import functools

import jax
import jax.numpy as jnp
from jax.experimental import pallas as pl
from jax.experimental.pallas import tpu as pltpu

_QT = 256
_BK = 2048
_TOPK = 5
_NEG = -3.0e38


def _topk_kernel(qn_ref, bank_ref, nrm_ref, vals_ref, idx_ref,
                 rv_ref, rn_ref, bn_ref, *, nk, K):
    k = pl.program_id(0)
    i = pl.program_id(1)
    rows = pl.ds(i * _QT, _QT)

    @pl.when(k == 0)
    def _init():
        rv_ref[rows, :] = jnp.full((_QT, rv_ref.shape[1]), _NEG, jnp.float32)
        rn_ref[rows, :] = jnp.full((_QT, rn_ref.shape[1]), _NEG, jnp.float32)

    @pl.when(i == 0)
    def _normalize():
        bn_ref[...] = bank_ref[...] / nrm_ref[...]

    s = jax.lax.dot_general(qn_ref[...], bn_ref[...], (((1,), (1,)), ((), ())),
                            preferred_element_type=jnp.float32)
    col = jax.lax.broadcasted_iota(jnp.int32, s.shape, 1) + k * _BK
    negcol = -col.astype(jnp.float32)
    s = jnp.where(col < K, s, _NEG)

    nsl = s.shape[1] // 128
    c0 = negcol[:, 0:128]
    vcols, ncols = [], []
    for j in range(_TOPK):
        V = s[:, 0:128]
        C = c0
        for t in range(1, nsl):
            st = s[:, t * 128:(t + 1) * 128]
            ct = c0 - float(128 * t)
            g = st > V
            V = jnp.where(g, st, V)
            C = jnp.where(g, ct, C)
        m = jnp.max(V, axis=1, keepdims=True)
        z = jnp.where(V == m, C, _NEG)
        ci = jnp.max(z, axis=1, keepdims=True)
        vcols.append(m)
        ncols.append(ci)
        if j < _TOPK - 1:
            s = jnp.where(negcol == ci, _NEG, s)

    cv = jnp.concatenate(vcols + [rv_ref[rows, :_TOPK]], axis=1)
    cn = jnp.concatenate(ncols + [rn_ref[rows, :_TOPK]], axis=1)
    nv, nn = [], []
    for j in range(_TOPK):
        m = jnp.max(cv, axis=1, keepdims=True)
        z = jnp.where(cv == m, cn, _NEG)
        ci = jnp.max(z, axis=1, keepdims=True)
        nv.append(m)
        nn.append(ci)
        cv = jnp.where(z == ci, _NEG, cv)
    pad = rv_ref.shape[1] - _TOPK
    fill = jnp.full((_QT, pad), _NEG, jnp.float32)
    rv_ref[rows, :] = jnp.concatenate(nv + [fill], axis=1)
    rn_ref[rows, :] = jnp.concatenate(nn + [fill], axis=1)

    @pl.when(k == nk - 1)
    def _emit():
        vals_ref[...] = rv_ref[rows, :_TOPK]
        idx_ref[...] = (-rn_ref[rows, :_TOPK]).astype(jnp.int32)


def _layer_norm(x, g, b, eps=1e-5):
    m = jnp.mean(x, axis=-1, keepdims=True)
    v = jnp.var(x, axis=-1, keepdims=True)
    return (x - m) / jnp.sqrt(v + eps) * g + b


def kernel(query, bank, W1, b1, g1, beta1, W2, b2, g2, beta2, top_k):
    Q, D = query.shape
    K = bank.shape[0]
    nq = Q // _QT
    nk = pl.cdiv(K, _BK)

    h = query @ W1 + b1
    h = _layer_norm(h, g1, beta1)
    h = jax.nn.relu(h)
    h = h @ W2 + b2
    q_emb = _layer_norm(h, g2, beta2)
    qn = q_emb / (jnp.linalg.norm(q_emb, axis=-1, keepdims=True) + 1e-8)
    nrm = jnp.linalg.norm(bank, axis=-1, keepdims=True) + 1e-8

    vals, idx = pl.pallas_call(
        functools.partial(_topk_kernel, nk=nk, K=K),
        grid=(nk, nq),
        in_specs=[
            pl.BlockSpec((_QT, D), lambda k, i: (i, 0)),
            pl.BlockSpec((_BK, D), lambda k, i: (k, 0)),
            pl.BlockSpec((_BK, 1), lambda k, i: (k, 0)),
        ],
        out_specs=[
            pl.BlockSpec((_QT, _TOPK), lambda k, i: (i, 0)),
            pl.BlockSpec((_QT, _TOPK), lambda k, i: (i, 0)),
        ],
        out_shape=[
            jax.ShapeDtypeStruct((Q, _TOPK), jnp.float32),
            jax.ShapeDtypeStruct((Q, _TOPK), jnp.int32),
        ],
        scratch_shapes=[
            pltpu.VMEM((Q, 128), jnp.float32),
            pltpu.VMEM((Q, 128), jnp.float32),
            pltpu.VMEM((_BK, D), jnp.float32),
        ],
        compiler_params=pltpu.CompilerParams(
            dimension_semantics=("arbitrary", "arbitrary"),
        ),
    )(qn, bank, nrm)
    return vals, idx


# speedup vs baseline: 1.2331x; 1.1174x over previous
import functools

import jax
import jax.numpy as jnp
from jax.experimental import pallas as pl
from jax.experimental.pallas import tpu as pltpu

_QT = 256
_BK = 4096
_TOPK = 5
_NEG = -3.0e38


def _topk_kernel(qn_ref, bank_ref, nrm_ref, vals_ref, idx_ref,
                 rv_ref, rn_ref, bn_ref, *, nk, K):
    k = pl.program_id(0)
    i = pl.program_id(1)
    rows = pl.ds(i * _QT, _QT)

    @pl.when(k == 0)
    def _init():
        rv_ref[rows, :] = jnp.full((_QT, rv_ref.shape[1]), _NEG, jnp.float32)
        rn_ref[rows, :] = jnp.full((_QT, rn_ref.shape[1]), _NEG, jnp.float32)

    @pl.when(i == 0)
    def _normalize():
        bn_ref[...] = bank_ref[...] / nrm_ref[...]

    s = jax.lax.dot_general(qn_ref[...], bn_ref[...], (((1,), (1,)), ((), ())),
                            preferred_element_type=jnp.float32)
    col = jax.lax.broadcasted_iota(jnp.int32, s.shape, 1) + k * _BK
    negcol = -col.astype(jnp.float32)
    s = jnp.where(col < K, s, _NEG)

    nsl = s.shape[1] // 128
    c0 = negcol[:, 0:128]
    vcols, ncols = [], []
    for j in range(_TOPK):
        V = s[:, 0:128]
        C = c0
        for t in range(1, nsl):
            st = s[:, t * 128:(t + 1) * 128]
            ct = c0 - float(128 * t)
            g = st > V
            V = jnp.where(g, st, V)
            C = jnp.where(g, ct, C)
        m = jnp.max(V, axis=1, keepdims=True)
        z = jnp.where(V == m, C, _NEG)
        ci = jnp.max(z, axis=1, keepdims=True)
        vcols.append(m)
        ncols.append(ci)
        if j < _TOPK - 1:
            s = jnp.where(negcol == ci, _NEG, s)

    cv = jnp.concatenate(vcols + [rv_ref[rows, :_TOPK]], axis=1)
    cn = jnp.concatenate(ncols + [rn_ref[rows, :_TOPK]], axis=1)
    nv, nn = [], []
    for j in range(_TOPK):
        m = jnp.max(cv, axis=1, keepdims=True)
        z = jnp.where(cv == m, cn, _NEG)
        ci = jnp.max(z, axis=1, keepdims=True)
        nv.append(m)
        nn.append(ci)
        cv = jnp.where(z == ci, _NEG, cv)
    pad = rv_ref.shape[1] - _TOPK
    fill = jnp.full((_QT, pad), _NEG, jnp.float32)
    rv_ref[rows, :] = jnp.concatenate(nv + [fill], axis=1)
    rn_ref[rows, :] = jnp.concatenate(nn + [fill], axis=1)

    @pl.when(k == nk - 1)
    def _emit():
        vals_ref[...] = rv_ref[rows, :_TOPK]
        idx_ref[...] = (-rn_ref[rows, :_TOPK]).astype(jnp.int32)


def _layer_norm(x, g, b, eps=1e-5):
    m = jnp.mean(x, axis=-1, keepdims=True)
    v = jnp.var(x, axis=-1, keepdims=True)
    return (x - m) / jnp.sqrt(v + eps) * g + b


def kernel(query, bank, W1, b1, g1, beta1, W2, b2, g2, beta2, top_k):
    Q, D = query.shape
    K = bank.shape[0]
    nq = Q // _QT
    nk = pl.cdiv(K, _BK)

    h = query @ W1 + b1
    h = _layer_norm(h, g1, beta1)
    h = jax.nn.relu(h)
    h = h @ W2 + b2
    q_emb = _layer_norm(h, g2, beta2)
    qn = q_emb / (jnp.linalg.norm(q_emb, axis=-1, keepdims=True) + 1e-8)
    nrm = jnp.linalg.norm(bank, axis=-1, keepdims=True) + 1e-8

    vals, idx = pl.pallas_call(
        functools.partial(_topk_kernel, nk=nk, K=K),
        grid=(nk, nq),
        in_specs=[
            pl.BlockSpec((_QT, D), lambda k, i: (i, 0)),
            pl.BlockSpec((_BK, D), lambda k, i: (k, 0)),
            pl.BlockSpec((_BK, 1), lambda k, i: (k, 0)),
        ],
        out_specs=[
            pl.BlockSpec((_QT, _TOPK), lambda k, i: (i, 0)),
            pl.BlockSpec((_QT, _TOPK), lambda k, i: (i, 0)),
        ],
        out_shape=[
            jax.ShapeDtypeStruct((Q, _TOPK), jnp.float32),
            jax.ShapeDtypeStruct((Q, _TOPK), jnp.int32),
        ],
        scratch_shapes=[
            pltpu.VMEM((Q, 128), jnp.float32),
            pltpu.VMEM((Q, 128), jnp.float32),
            pltpu.VMEM((_BK, D), jnp.float32),
        ],
        compiler_params=pltpu.CompilerParams(
            dimension_semantics=("arbitrary", "arbitrary"),
        ),
    )(qn, bank, nrm)
    return vals, idx


# speedup vs baseline: 1.3403x; 1.0870x over previous
import functools

import jax
import jax.numpy as jnp
from jax.experimental import pallas as pl
from jax.experimental.pallas import tpu as pltpu

_QT = 512
_BK = 4096
_TOPK = 5
_NEG = -3.0e38


def _topk_kernel(qn_ref, bank_ref, nrm_ref, vals_ref, idx_ref,
                 rv_ref, rn_ref, bn_ref, *, nk, K):
    k = pl.program_id(0)
    i = pl.program_id(1)
    rows = pl.ds(i * _QT, _QT)

    @pl.when(k == 0)
    def _init():
        rv_ref[rows, :] = jnp.full((_QT, rv_ref.shape[1]), _NEG, jnp.float32)
        rn_ref[rows, :] = jnp.full((_QT, rn_ref.shape[1]), _NEG, jnp.float32)

    @pl.when(i == 0)
    def _normalize():
        bn_ref[...] = bank_ref[...] / nrm_ref[...]

    s = jax.lax.dot_general(qn_ref[...], bn_ref[...], (((1,), (1,)), ((), ())),
                            preferred_element_type=jnp.float32)
    col = jax.lax.broadcasted_iota(jnp.int32, s.shape, 1) + k * _BK
    negcol = -col.astype(jnp.float32)
    s = jnp.where(col < K, s, _NEG)

    nsl = s.shape[1] // 128
    c0 = negcol[:, 0:128]
    vcols, ncols = [], []
    for j in range(_TOPK):
        V = s[:, 0:128]
        C = c0
        for t in range(1, nsl):
            st = s[:, t * 128:(t + 1) * 128]
            ct = c0 - float(128 * t)
            g = st > V
            V = jnp.where(g, st, V)
            C = jnp.where(g, ct, C)
        m = jnp.max(V, axis=1, keepdims=True)
        z = jnp.where(V == m, C, _NEG)
        ci = jnp.max(z, axis=1, keepdims=True)
        vcols.append(m)
        ncols.append(ci)
        if j < _TOPK - 1:
            s = jnp.where(negcol == ci, _NEG, s)

    cv = jnp.concatenate(vcols + [rv_ref[rows, :_TOPK]], axis=1)
    cn = jnp.concatenate(ncols + [rn_ref[rows, :_TOPK]], axis=1)
    nv, nn = [], []
    for j in range(_TOPK):
        m = jnp.max(cv, axis=1, keepdims=True)
        z = jnp.where(cv == m, cn, _NEG)
        ci = jnp.max(z, axis=1, keepdims=True)
        nv.append(m)
        nn.append(ci)
        cv = jnp.where(z == ci, _NEG, cv)
    pad = rv_ref.shape[1] - _TOPK
    fill = jnp.full((_QT, pad), _NEG, jnp.float32)
    rv_ref[rows, :] = jnp.concatenate(nv + [fill], axis=1)
    rn_ref[rows, :] = jnp.concatenate(nn + [fill], axis=1)

    @pl.when(k == nk - 1)
    def _emit():
        vals_ref[...] = rv_ref[rows, :_TOPK]
        idx_ref[...] = (-rn_ref[rows, :_TOPK]).astype(jnp.int32)


def _layer_norm(x, g, b, eps=1e-5):
    m = jnp.mean(x, axis=-1, keepdims=True)
    v = jnp.var(x, axis=-1, keepdims=True)
    return (x - m) / jnp.sqrt(v + eps) * g + b


def kernel(query, bank, W1, b1, g1, beta1, W2, b2, g2, beta2, top_k):
    Q, D = query.shape
    K = bank.shape[0]
    nq = Q // _QT
    nk = pl.cdiv(K, _BK)

    h = query @ W1 + b1
    h = _layer_norm(h, g1, beta1)
    h = jax.nn.relu(h)
    h = h @ W2 + b2
    q_emb = _layer_norm(h, g2, beta2)
    qn = q_emb / (jnp.linalg.norm(q_emb, axis=-1, keepdims=True) + 1e-8)
    nrm = jnp.linalg.norm(bank, axis=-1, keepdims=True) + 1e-8

    vals, idx = pl.pallas_call(
        functools.partial(_topk_kernel, nk=nk, K=K),
        grid=(nk, nq),
        in_specs=[
            pl.BlockSpec((_QT, D), lambda k, i: (i, 0)),
            pl.BlockSpec((_BK, D), lambda k, i: (k, 0)),
            pl.BlockSpec((_BK, 1), lambda k, i: (k, 0)),
        ],
        out_specs=[
            pl.BlockSpec((_QT, _TOPK), lambda k, i: (i, 0)),
            pl.BlockSpec((_QT, _TOPK), lambda k, i: (i, 0)),
        ],
        out_shape=[
            jax.ShapeDtypeStruct((Q, _TOPK), jnp.float32),
            jax.ShapeDtypeStruct((Q, _TOPK), jnp.int32),
        ],
        scratch_shapes=[
            pltpu.VMEM((Q, 128), jnp.float32),
            pltpu.VMEM((Q, 128), jnp.float32),
            pltpu.VMEM((_BK, D), jnp.float32),
        ],
        compiler_params=pltpu.CompilerParams(
            dimension_semantics=("arbitrary", "arbitrary"),
        ),
    )(qn, bank, nrm)
    return vals, idx


# speedup vs baseline: 1.3989x; 1.0438x over previous
import functools

import jax
import jax.numpy as jnp
from jax.experimental import pallas as pl
from jax.experimental.pallas import tpu as pltpu

_QT = 1024
_BK = 4096
_TOPK = 5
_NEG = -3.0e38


def _topk_kernel(qn_ref, bank_ref, nrm_ref, vals_ref, idx_ref,
                 rv_ref, rn_ref, bn_ref, *, nk, K):
    k = pl.program_id(0)
    i = pl.program_id(1)
    rows = pl.ds(i * _QT, _QT)

    @pl.when(k == 0)
    def _init():
        rv_ref[rows, :] = jnp.full((_QT, rv_ref.shape[1]), _NEG, jnp.float32)
        rn_ref[rows, :] = jnp.full((_QT, rn_ref.shape[1]), _NEG, jnp.float32)

    @pl.when(i == 0)
    def _normalize():
        bn_ref[...] = bank_ref[...] / nrm_ref[...]

    s = jax.lax.dot_general(qn_ref[...], bn_ref[...], (((1,), (1,)), ((), ())),
                            preferred_element_type=jnp.float32)
    col = jax.lax.broadcasted_iota(jnp.int32, s.shape, 1) + k * _BK
    negcol = -col.astype(jnp.float32)
    s = jnp.where(col < K, s, _NEG)

    nsl = s.shape[1] // 128
    c0 = negcol[:, 0:128]
    vcols, ncols = [], []
    for j in range(_TOPK):
        V = s[:, 0:128]
        C = c0
        for t in range(1, nsl):
            st = s[:, t * 128:(t + 1) * 128]
            ct = c0 - float(128 * t)
            g = st > V
            V = jnp.where(g, st, V)
            C = jnp.where(g, ct, C)
        m = jnp.max(V, axis=1, keepdims=True)
        z = jnp.where(V == m, C, _NEG)
        ci = jnp.max(z, axis=1, keepdims=True)
        vcols.append(m)
        ncols.append(ci)
        if j < _TOPK - 1:
            s = jnp.where(negcol == ci, _NEG, s)

    cv = jnp.concatenate(vcols + [rv_ref[rows, :_TOPK]], axis=1)
    cn = jnp.concatenate(ncols + [rn_ref[rows, :_TOPK]], axis=1)
    nv, nn = [], []
    for j in range(_TOPK):
        m = jnp.max(cv, axis=1, keepdims=True)
        z = jnp.where(cv == m, cn, _NEG)
        ci = jnp.max(z, axis=1, keepdims=True)
        nv.append(m)
        nn.append(ci)
        cv = jnp.where(z == ci, _NEG, cv)
    pad = rv_ref.shape[1] - _TOPK
    fill = jnp.full((_QT, pad), _NEG, jnp.float32)
    rv_ref[rows, :] = jnp.concatenate(nv + [fill], axis=1)
    rn_ref[rows, :] = jnp.concatenate(nn + [fill], axis=1)

    @pl.when(k == nk - 1)
    def _emit():
        vals_ref[...] = rv_ref[rows, :_TOPK]
        idx_ref[...] = (-rn_ref[rows, :_TOPK]).astype(jnp.int32)


def _layer_norm(x, g, b, eps=1e-5):
    m = jnp.mean(x, axis=-1, keepdims=True)
    v = jnp.var(x, axis=-1, keepdims=True)
    return (x - m) / jnp.sqrt(v + eps) * g + b


def kernel(query, bank, W1, b1, g1, beta1, W2, b2, g2, beta2, top_k):
    Q, D = query.shape
    K = bank.shape[0]
    nq = Q // _QT
    nk = pl.cdiv(K, _BK)

    h = query @ W1 + b1
    h = _layer_norm(h, g1, beta1)
    h = jax.nn.relu(h)
    h = h @ W2 + b2
    q_emb = _layer_norm(h, g2, beta2)
    qn = q_emb / (jnp.linalg.norm(q_emb, axis=-1, keepdims=True) + 1e-8)
    nrm = jnp.linalg.norm(bank, axis=-1, keepdims=True) + 1e-8

    vals, idx = pl.pallas_call(
        functools.partial(_topk_kernel, nk=nk, K=K),
        grid=(nk, nq),
        in_specs=[
            pl.BlockSpec((_QT, D), lambda k, i: (i, 0)),
            pl.BlockSpec((_BK, D), lambda k, i: (k, 0)),
            pl.BlockSpec((_BK, 1), lambda k, i: (k, 0)),
        ],
        out_specs=[
            pl.BlockSpec((_QT, _TOPK), lambda k, i: (i, 0)),
            pl.BlockSpec((_QT, _TOPK), lambda k, i: (i, 0)),
        ],
        out_shape=[
            jax.ShapeDtypeStruct((Q, _TOPK), jnp.float32),
            jax.ShapeDtypeStruct((Q, _TOPK), jnp.int32),
        ],
        scratch_shapes=[
            pltpu.VMEM((Q, 128), jnp.float32),
            pltpu.VMEM((Q, 128), jnp.float32),
            pltpu.VMEM((_BK, D), jnp.float32),
        ],
        compiler_params=pltpu.CompilerParams(
            dimension_semantics=("arbitrary", "arbitrary"),
        ),
    )(qn, bank, nrm)
    return vals, idx
